# Initial kernel scaffold; baseline (speedup 1.0000x reference)
import jax
import jax.numpy as jnp
from jax.experimental import pallas as pl


def kernel(x, w_t, b_row):
    raise NotImplementedError("write your pallas kernel here")


# speedup vs baseline: 1.1251x; 1.1251x over previous
import functools

import jax
import jax.numpy as jnp
from jax.experimental import pallas as pl
from jax.experimental.pallas import tpu as pltpu


def _round_up(a: int, b: int) -> int:
    return (a + b - 1) // b * b


def _fc_kernel(x_ref, w_ref, b_ref, o_ref):
    y = jnp.dot(x_ref[...], w_ref[...], preferred_element_type=jnp.float32)
    o_ref[...] = jnp.maximum(y + b_ref[...], 0.0).astype(o_ref.dtype)


@functools.partial(jax.jit, static_argnames=("tm",))
def _fc_apply(x, w_t, b_row, tm: int):
    n = x.shape[0]
    kin, h = w_t.shape
    out_dtype = x.dtype

    x_flat = x.reshape(n, kin)
    n_pad = _round_up(n, tm)
    if n_pad != n:
        x_flat = jnp.pad(x_flat, ((0, n_pad - n), (0, 0)))

    grid = (n_pad // tm,)

    in_itemsize = jnp.dtype(x_flat.dtype).itemsize
    cost = pl.CostEstimate(
        flops=2 * n_pad * kin * h,
        transcendentals=0,
        bytes_accessed=(n_pad * kin * in_itemsize
                        + 2 * kin * h * in_itemsize
                        + h * 4
                        + n_pad * h * 4),
    )

    out = pl.pallas_call(
        _fc_kernel,
        out_shape=jax.ShapeDtypeStruct((n_pad, h), out_dtype),
        grid=grid,
        in_specs=[
            pl.BlockSpec((tm, kin), lambda i: (i, 0)),
            pl.BlockSpec((kin, h), lambda i: (0, 0)),
            pl.BlockSpec((1, h), lambda i: (0, 0)),
        ],
        out_specs=pl.BlockSpec((tm, h), lambda i: (i, 0)),
        compiler_params=pltpu.CompilerParams(
            dimension_semantics=("parallel",),
            vmem_limit_bytes=60 * 1024 * 1024,
        ),
        cost_estimate=cost,
    )(x_flat, w_t, b_row)

    if n_pad != n:
        out = out[:n]
    return out


def kernel(x, w_t, b_row):
    n = x.shape[0]
    kin, h = w_t.shape
    tm = 512 if n >= 1024 else max(8, _round_up(n // 2, 8))
    return _fc_apply(x, w_t, b_row, tm)


# speedup vs baseline: 1.8007x; 1.6004x over previous
import functools

import jax
import jax.numpy as jnp
from jax.experimental import pallas as pl
from jax.experimental.pallas import tpu as pltpu


def _round_up(a: int, b: int) -> int:
    return (a + b - 1) // b * b


def _make_fc_kernel(tm: int, k: int, h: int):
    def _fc_kernel(x_hbm, w_ref, b_ref, o_ref, xbuf, sems):
        base = pl.program_id(0) * tm

        def cp(j):
            return pltpu.make_async_copy(
                x_hbm.at[pl.ds(base, tm), j, :], xbuf.at[j], sems.at[j])

        for j in range(k):
            cp(j).start()
        y = None
        for j in range(k):
            cp(j).wait()
            d = jnp.dot(xbuf[j], w_ref[j * h:(j + 1) * h, :],
                        preferred_element_type=jnp.float32)
            y = d if y is None else y + d
        o_ref[...] = jnp.maximum(y + b_ref[...], 0.0).astype(o_ref.dtype)

    return _fc_kernel


@functools.partial(jax.jit, static_argnames=("tm",))
def _fc_apply(x, w_t, b_row, tm: int):
    n, k, h = x.shape
    kin = w_t.shape[0]
    out_dtype = x.dtype

    n_pad = _round_up(n, tm)
    if n_pad != n:
        x = jnp.pad(x, ((0, n_pad - n), (0, 0), (0, 0)))

    grid = (n_pad // tm,)

    cost = pl.CostEstimate(
        flops=2 * n_pad * kin * h,
        transcendentals=0,
        bytes_accessed=(n_pad * kin * 4 + 2 * kin * h * 4 + h * 4
                        + n_pad * h * 4),
    )

    out = pl.pallas_call(
        _make_fc_kernel(tm, k, h),
        out_shape=jax.ShapeDtypeStruct((n_pad, h), out_dtype),
        grid=grid,
        in_specs=[pl.BlockSpec(memory_space=pltpu.MemorySpace.HBM),
                  pl.BlockSpec((kin, h), lambda i: (0, 0)),
                  pl.BlockSpec((1, h), lambda i: (0, 0))],
        out_specs=pl.BlockSpec((tm, h), lambda i: (i, 0)),
        scratch_shapes=[pltpu.VMEM((k, tm, h), jnp.float32),
                        pltpu.SemaphoreType.DMA((k,))],
        compiler_params=pltpu.CompilerParams(
            dimension_semantics=("parallel",),
            vmem_limit_bytes=60 * 1024 * 1024,
        ),
        cost_estimate=cost,
    )(x, w_t, b_row)

    if n_pad != n:
        out = out[:n]
    return out


def kernel(x, w_t, b_row):
    n = x.shape[0]
    tm = 512 if n >= 1024 else max(8, _round_up(n // 2, 8))
    return _fc_apply(x, w_t, b_row, tm)
